# baseline (speedup 1.0000x reference)
import functools

import jax
import jax.numpy as jnp
from jax import lax
from jax.experimental import pallas as pl
from jax.experimental.pallas import tpu as pltpu
from jax.experimental.pallas import tpu_sc as plsc

TOP_K = 2
BT = 256


def _round_bf16_bits(v):
    bits = lax.bitcast_convert_type(v, jnp.int32)
    lsb = lax.shift_right_logical(bits, 16) & 1
    rounded = bits + 0x7FFF + lsb
    return lax.shift_right_logical(rounded, 16)


def _router_body(hs_ref, gw_ref, i1_ref, i2_ref, wa_ref, wb_ref, xp_ref):
    x = hs_ref[...]
    half = x.shape[1] // 2
    lo = _round_bf16_bits(x[:, :half])
    hi = lax.shift_left(_round_bf16_bits(x[:, half:]), 16)
    xp_ref[...] = lo | hi
    gw = gw_ref[...]
    logits = lax.dot_general(x, gw, (((1,), (1,)), ((), ())),
                             preferred_element_type=jnp.float32)
    tb, e = logits.shape
    iota = lax.broadcasted_iota(jnp.int32, (tb, e), 1)
    m1 = jnp.max(logits, axis=1, keepdims=True)
    i1 = jnp.min(jnp.where(logits == m1, iota, e), axis=1, keepdims=True)
    masked = jnp.where(iota == i1, -jnp.inf, logits)
    m2 = jnp.max(masked, axis=1, keepdims=True)
    i2 = jnp.min(jnp.where(masked == m2, iota, e), axis=1, keepdims=True)
    wa = jax.nn.sigmoid(m1 - m2)
    i1_ref[...] = i1
    i2_ref[...] = i2
    wa_ref[...] = wa
    wb_ref[...] = 1.0 - wa


def _router(hs, gw):
    T, D = hs.shape
    E = gw.shape[0]
    TB = 256
    grid = (T // TB,)
    out_shape = (
        jax.ShapeDtypeStruct((T, 1), jnp.int32),
        jax.ShapeDtypeStruct((T, 1), jnp.int32),
        jax.ShapeDtypeStruct((T, 1), jnp.float32),
        jax.ShapeDtypeStruct((T, 1), jnp.float32),
        jax.ShapeDtypeStruct((T, D // 2), jnp.int32),
    )
    spec1 = pl.BlockSpec((TB, 1), lambda g: (g, 0))
    return pl.pallas_call(
        _router_body,
        grid=grid,
        in_specs=[
            pl.BlockSpec((TB, D), lambda g: (g, 0)),
            pl.BlockSpec((E, D), lambda g: (0, 0)),
        ],
        out_specs=(spec1, spec1, spec1, spec1,
                   pl.BlockSpec((TB, D // 2), lambda g: (g, 0))),
        out_shape=out_shape,
    )(hs, gw)


def _metadata(i1, i2, wa, wb, E, Gmax, S):
    T = i1.shape[0]
    ex = jnp.stack([i1, i2], axis=1).reshape(-1)
    wx = jnp.stack([wa, wb], axis=1).reshape(-1)
    onehot = (ex[:, None] == jnp.arange(E)[None, :]).astype(jnp.int32)
    incl = jnp.cumsum(onehot, axis=0)
    counts = incl[-1]
    rank = jnp.take_along_axis(incl, ex[:, None], axis=1)[:, 0] - 1
    padded = ((counts + BT - 1) // BT) * BT
    seg_end = jnp.cumsum(padded)
    seg_start = seg_end - padded
    dest = seg_start[ex] + rank
    token_of_slot = jnp.zeros((S,), jnp.int32).at[dest].set(
        jnp.arange(2 * T, dtype=jnp.int32) // 2)
    weight_of_slot = jnp.zeros((S,), jnp.float32).at[dest].set(wx)
    p0 = dest[0::2].astype(jnp.int32)
    p1 = dest[1::2].astype(jnp.int32)
    n_active = (seg_end[-1] // BT).astype(jnp.int32)
    blk = jnp.arange(Gmax, dtype=jnp.int32) * BT
    block_expert = jnp.minimum(
        jnp.searchsorted(seg_end, blk, side='right'), E - 1).astype(jnp.int32)
    return (token_of_slot, weight_of_slot, p0, p1, block_expert,
            n_active[None])


def _sc_gather(hs_rows, token_of_slot, S):
    T, W = hs_rows.shape
    info = plsc.get_sparse_core_info()
    NW = info.num_cores * info.num_subcores
    per_w = S // NW
    NB = 4
    CH = 24
    n_ch = per_w // CH
    mesh = plsc.VectorSubcoreMesh(core_axis_name="c", subcore_axis_name="s")

    @functools.partial(
        pl.kernel, mesh=mesh,
        out_type=jax.ShapeDtypeStruct((S, W), hs_rows.dtype),
        scratch_types=(
            [pltpu.VMEM((per_w,), jnp.int32)]
            + [pltpu.VMEM((CH, W), hs_rows.dtype) for _ in range(NB)]
            + [pltpu.SemaphoreType.DMA for _ in range(2 * NB)]
        ),
    )
    def gather_k(hs_hbm, tos_hbm, out_hbm, idx_v, *scr):
        bufs = list(scr[:NB])
        sems = list(scr[NB:2 * NB])
        wsems = list(scr[2 * NB:])
        nc = info.num_cores
        wid = lax.axis_index("s") * nc + lax.axis_index("c")
        base = wid * per_w
        pltpu.sync_copy(tos_hbm.at[pl.ds(base, per_w)], idx_v)
        handles = [None] * NB
        writes = [None] * NB
        for k in range(min(NB, n_ch)):
            handles[k] = pltpu.async_copy(
                hs_hbm.at[idx_v.at[pl.ds(k * CH, CH)]], bufs[k], sems[k])
        for c in range(n_ch):
            j = c % NB
            handles[j].wait()
            writes[j] = pltpu.async_copy(
                bufs[j], out_hbm.at[pl.ds(base + c * CH, CH)], wsems[j])
            if c + NB < n_ch:
                writes[j].wait()
                writes[j] = None
                handles[j] = pltpu.async_copy(
                    hs_hbm.at[idx_v.at[pl.ds((c + NB) * CH, CH)]],
                    bufs[j], sems[j])
        for wh in writes:
            if wh is not None:
                wh.wait()

    return gather_k(hs_rows, token_of_slot)


def _ffn_body(be_ref, na_ref, x_ref, w1_ref, w3_ref, w2_ref, ws_ref, y_ref,
              w1c_ref, w3c_ref, w2c_ref, last_ref):
    g = pl.program_id(0)

    @pl.when(g < na_ref[0])
    def _():
        e = be_ref[g]

        @pl.when((g == 0) | (e != last_ref[0]))
        def _():
            w1c_ref[...] = w1_ref[0].astype(jnp.bfloat16)
            w3c_ref[...] = w3_ref[0].astype(jnp.bfloat16)
            w2c_ref[...] = w2_ref[0].astype(jnp.bfloat16)
            last_ref[0] = e

        xi = x_ref[...]
        half = xi.shape[1]
        xa = lax.bitcast_convert_type(
            lax.shift_left(xi, 16), jnp.float32).astype(jnp.bfloat16)
        xb = lax.bitcast_convert_type(
            xi & jnp.int32(-65536), jnp.float32).astype(jnp.bfloat16)
        dn = (((1,), (1,)), ((), ()))
        a = (lax.dot_general(xa, w1c_ref[:, :half], dn,
                             preferred_element_type=jnp.float32) +
             lax.dot_general(xb, w1c_ref[:, half:], dn,
                             preferred_element_type=jnp.float32))
        b = (lax.dot_general(xa, w3c_ref[:, :half], dn,
                             preferred_element_type=jnp.float32) +
             lax.dot_general(xb, w3c_ref[:, half:], dn,
                             preferred_element_type=jnp.float32))
        h = ((a * jax.nn.sigmoid(a)) * b).astype(jnp.bfloat16)
        y = lax.dot_general(h, w2c_ref[...], dn,
                            preferred_element_type=jnp.float32)
        y_ref[...] = y * ws_ref[...]


def _ffn(xg, w1, w3, w2, weight_of_slot, block_expert, n_active, Gmax, S):
    E, F, D = w1.shape
    ws2d = weight_of_slot.reshape(S, 1)
    grid_spec = pltpu.PrefetchScalarGridSpec(
        num_scalar_prefetch=2,
        grid=(Gmax,),
        in_specs=[
            pl.BlockSpec((BT, D // 2), lambda g, be, na: (g, 0)),
            pl.BlockSpec((1, F, D), lambda g, be, na: (be[g], 0, 0)),
            pl.BlockSpec((1, F, D), lambda g, be, na: (be[g], 0, 0)),
            pl.BlockSpec((1, D, F), lambda g, be, na: (be[g], 0, 0)),
            pl.BlockSpec((BT, 1), lambda g, be, na: (g, 0)),
        ],
        out_specs=pl.BlockSpec((BT, D), lambda g, be, na: (g, 0)),
        scratch_shapes=[
            pltpu.VMEM((F, D), jnp.bfloat16),
            pltpu.VMEM((F, D), jnp.bfloat16),
            pltpu.VMEM((D, F), jnp.bfloat16),
            pltpu.SMEM((1,), jnp.int32),
        ],
    )
    return pl.pallas_call(
        _ffn_body,
        grid_spec=grid_spec,
        out_shape=jax.ShapeDtypeStruct((S, D), jnp.float32),
    )(block_expert, n_active, xg, w1, w3, w2, ws2d)


def _sc_combine(yg, p0, p1, T, D):
    info = plsc.get_sparse_core_info()
    NW = info.num_cores * info.num_subcores
    L = info.num_lanes
    per_w = T // NW
    CH = 8
    n_ch = per_w // CH
    mesh = plsc.VectorSubcoreMesh(core_axis_name="c", subcore_axis_name="s")

    @functools.partial(
        pl.kernel, mesh=mesh,
        out_type=jax.ShapeDtypeStruct((T, D), jnp.float32),
        scratch_types=[
            pltpu.VMEM((per_w,), jnp.int32),
            pltpu.VMEM((per_w,), jnp.int32),
            pltpu.VMEM((CH, D), jnp.float32),
            pltpu.VMEM((CH, D), jnp.float32),
            pltpu.VMEM((CH, D), jnp.float32),
            pltpu.VMEM((CH, D), jnp.float32),
            pltpu.SemaphoreType.DMA,
            pltpu.SemaphoreType.DMA,
            pltpu.SemaphoreType.DMA,
            pltpu.SemaphoreType.DMA,
        ],
    )
    def combine_k(yg_hbm, p0_hbm, p1_hbm, out_hbm, p0_v, p1_v,
                  ba0, bb0, ba1, bb1, sa0, sb0, sa1, sb1):
        nc = info.num_cores
        wid = lax.axis_index("s") * nc + lax.axis_index("c")
        base = wid * per_w
        pltpu.sync_copy(p0_hbm.at[pl.ds(base, per_w)], p0_v)
        pltpu.sync_copy(p1_hbm.at[pl.ds(base, per_w)], p1_v)
        bas, bbs = [ba0, ba1], [bb0, bb1]
        sas, sbs = [sa0, sa1], [sb0, sb1]
        ha, hb = [None, None], [None, None]

        def issue(c):
            j = c % 2
            ha[j] = pltpu.async_copy(
                yg_hbm.at[p0_v.at[pl.ds(c * CH, CH)]], bas[j], sas[j])
            hb[j] = pltpu.async_copy(
                yg_hbm.at[p1_v.at[pl.ds(c * CH, CH)]], bbs[j], sbs[j])

        issue(0)
        for c in range(n_ch):
            j = c % 2
            if c + 1 < n_ch:
                issue(c + 1)
            ha[j].wait()
            hb[j].wait()
            for r in range(CH):
                def add_row(k, _, r=r, j=j):
                    sl = pl.ds(k * L, L)
                    bas[j][r, sl] = bas[j][r, sl] + bbs[j][r, sl]
                    return 0
                lax.fori_loop(0, D // L, add_row, 0, unroll=8)
            pltpu.sync_copy(bas[j], out_hbm.at[pl.ds(base + c * CH, CH)])

    return combine_k(yg, p0, p1)


def kernel(hidden_states, gate_weight, w1, w3, w2):
    T, D = hidden_states.shape
    E = gate_weight.shape[0]
    Gmax = (TOP_K * T) // BT + E
    S = Gmax * BT

    i1, i2, wa, wb, xpack = _router(hidden_states, gate_weight)
    (token_of_slot, weight_of_slot, p0, p1, block_expert,
     n_active) = _metadata(i1[:, 0], i2[:, 0], wa[:, 0], wb[:, 0], E, Gmax, S)
    xg = _sc_gather(xpack, token_of_slot, S)
    yg = _ffn(xg, w1, w3, w2, weight_of_slot, block_expert,
              n_active, Gmax, S)
    return _sc_combine(yg, p0, p1, T, D)


# speedup vs baseline: 1.0423x; 1.0423x over previous
import functools

import jax
import jax.numpy as jnp
from jax import lax
from jax.experimental import pallas as pl
from jax.experimental.pallas import tpu as pltpu
from jax.experimental.pallas import tpu_sc as plsc

TOP_K = 2
BT = 256


def _round_bf16_bits(v):
    bits = lax.bitcast_convert_type(v, jnp.int32)
    lsb = lax.shift_right_logical(bits, 16) & 1
    rounded = bits + 0x7FFF + lsb
    return lax.shift_right_logical(rounded, 16)


def _router_body(hs_ref, gw_ref, i1_ref, i2_ref, wa_ref, wb_ref, xp_ref):
    x = hs_ref[...]
    half = x.shape[1] // 2
    lo = _round_bf16_bits(x[:, :half])
    hi = lax.shift_left(_round_bf16_bits(x[:, half:]), 16)
    xp_ref[...] = lo | hi
    gw = gw_ref[...]
    logits = lax.dot_general(x, gw, (((1,), (1,)), ((), ())),
                             preferred_element_type=jnp.float32)
    tb, e = logits.shape
    iota = lax.broadcasted_iota(jnp.int32, (tb, e), 1)
    m1 = jnp.max(logits, axis=1, keepdims=True)
    i1 = jnp.min(jnp.where(logits == m1, iota, e), axis=1, keepdims=True)
    masked = jnp.where(iota == i1, -jnp.inf, logits)
    m2 = jnp.max(masked, axis=1, keepdims=True)
    i2 = jnp.min(jnp.where(masked == m2, iota, e), axis=1, keepdims=True)
    wa = jax.nn.sigmoid(m1 - m2)
    i1_ref[...] = i1
    i2_ref[...] = i2
    wa_ref[...] = wa
    wb_ref[...] = 1.0 - wa


def _router(hs, gw):
    T, D = hs.shape
    E = gw.shape[0]
    TB = 256
    grid = (T // TB,)
    out_shape = (
        jax.ShapeDtypeStruct((T, 1), jnp.int32),
        jax.ShapeDtypeStruct((T, 1), jnp.int32),
        jax.ShapeDtypeStruct((T, 1), jnp.float32),
        jax.ShapeDtypeStruct((T, 1), jnp.float32),
        jax.ShapeDtypeStruct((T, D // 2), jnp.int32),
    )
    spec1 = pl.BlockSpec((TB, 1), lambda g: (g, 0))
    return pl.pallas_call(
        _router_body,
        grid=grid,
        in_specs=[
            pl.BlockSpec((TB, D), lambda g: (g, 0)),
            pl.BlockSpec((E, D), lambda g: (0, 0)),
        ],
        out_specs=(spec1, spec1, spec1, spec1,
                   pl.BlockSpec((TB, D // 2), lambda g: (g, 0))),
        out_shape=out_shape,
    )(hs, gw)


def _metadata(i1, i2, wa, wb, E, Gmax, S):
    T = i1.shape[0]
    ex = jnp.stack([i1, i2], axis=1).reshape(-1)
    wx = jnp.stack([wa, wb], axis=1).reshape(-1)
    onehot = (ex[:, None] == jnp.arange(E)[None, :]).astype(jnp.int32)
    incl = jnp.cumsum(onehot, axis=0)
    counts = incl[-1]
    rank = jnp.take_along_axis(incl, ex[:, None], axis=1)[:, 0] - 1
    padded = ((counts + BT - 1) // BT) * BT
    seg_end = jnp.cumsum(padded)
    seg_start = seg_end - padded
    dest = seg_start[ex] + rank
    token_of_slot = jnp.zeros((S,), jnp.int32).at[dest].set(
        jnp.arange(2 * T, dtype=jnp.int32) // 2)
    weight_of_slot = jnp.zeros((S,), jnp.float32).at[dest].set(wx)
    p0 = dest[0::2].astype(jnp.int32)
    p1 = dest[1::2].astype(jnp.int32)
    n_active = (seg_end[-1] // BT).astype(jnp.int32)
    blk = jnp.arange(Gmax, dtype=jnp.int32) * BT
    block_expert = jnp.minimum(
        jnp.searchsorted(seg_end, blk, side='right'), E - 1).astype(jnp.int32)
    return (token_of_slot, weight_of_slot, p0, p1, block_expert,
            n_active[None])


def _sc_gather(hs_rows, token_of_slot, S):
    T, W = hs_rows.shape
    info = plsc.get_sparse_core_info()
    NW = info.num_cores * info.num_subcores
    per_w = S // NW
    CH = 48
    n_ch = per_w // CH
    mesh = plsc.VectorSubcoreMesh(core_axis_name="c", subcore_axis_name="s")

    @functools.partial(
        pl.kernel, mesh=mesh,
        out_type=jax.ShapeDtypeStruct((S, W), hs_rows.dtype),
        scratch_types=[
            pltpu.VMEM((per_w,), jnp.int32),
            pltpu.VMEM((CH, W), hs_rows.dtype),
            pltpu.VMEM((CH, W), hs_rows.dtype),
            pltpu.SemaphoreType.DMA,
            pltpu.SemaphoreType.DMA,
        ],
    )
    def gather_k(hs_hbm, tos_hbm, out_hbm, idx_v, buf0, buf1, sem0, sem1):
        nc = info.num_cores
        wid = lax.axis_index("s") * nc + lax.axis_index("c")
        base = wid * per_w
        pltpu.sync_copy(tos_hbm.at[pl.ds(base, per_w)], idx_v)
        bufs, sems = [buf0, buf1], [sem0, sem1]
        handles = [None, None]
        handles[0] = pltpu.async_copy(
            hs_hbm.at[idx_v.at[pl.ds(0, CH)]], bufs[0], sems[0])
        for c in range(n_ch):
            if c + 1 < n_ch:
                j = (c + 1) % 2
                handles[j] = pltpu.async_copy(
                    hs_hbm.at[idx_v.at[pl.ds((c + 1) * CH, CH)]],
                    bufs[j], sems[j])
            handles[c % 2].wait()
            pltpu.sync_copy(bufs[c % 2], out_hbm.at[pl.ds(base + c * CH, CH)])

    return gather_k(hs_rows, token_of_slot)


def _ffn_body(be_ref, na_ref, x_ref, w1_ref, w3_ref, w2_ref, ws_ref, y_ref):
    g = pl.program_id(0)

    @pl.when(g < na_ref[0])
    def _():
        xi = x_ref[...]
        half = xi.shape[1]
        xa = lax.bitcast_convert_type(
            lax.shift_left(xi, 16), jnp.float32).astype(jnp.bfloat16)
        xb = lax.bitcast_convert_type(
            xi & jnp.int32(-65536), jnp.float32).astype(jnp.bfloat16)
        w1b = w1_ref[0].astype(jnp.bfloat16)
        w3b = w3_ref[0].astype(jnp.bfloat16)
        w2b = w2_ref[0].astype(jnp.bfloat16)
        dn = (((1,), (1,)), ((), ()))
        a = (lax.dot_general(xa, w1b[:, :half], dn,
                             preferred_element_type=jnp.float32) +
             lax.dot_general(xb, w1b[:, half:], dn,
                             preferred_element_type=jnp.float32))
        b = (lax.dot_general(xa, w3b[:, :half], dn,
                             preferred_element_type=jnp.float32) +
             lax.dot_general(xb, w3b[:, half:], dn,
                             preferred_element_type=jnp.float32))
        h = ((a * jax.nn.sigmoid(a)) * b).astype(jnp.bfloat16)
        y = lax.dot_general(h, w2b, dn,
                            preferred_element_type=jnp.float32)
        y_ref[...] = y * ws_ref[...]


def _ffn(xg, w1, w3, w2, weight_of_slot, block_expert, n_active, Gmax, S):
    E, F, D = w1.shape
    ws2d = weight_of_slot.reshape(S, 1)
    grid_spec = pltpu.PrefetchScalarGridSpec(
        num_scalar_prefetch=2,
        grid=(Gmax,),
        in_specs=[
            pl.BlockSpec((BT, D // 2), lambda g, be, na: (g, 0)),
            pl.BlockSpec((1, F, D), lambda g, be, na: (be[g], 0, 0)),
            pl.BlockSpec((1, F, D), lambda g, be, na: (be[g], 0, 0)),
            pl.BlockSpec((1, D, F), lambda g, be, na: (be[g], 0, 0)),
            pl.BlockSpec((BT, 1), lambda g, be, na: (g, 0)),
        ],
        out_specs=pl.BlockSpec((BT, D), lambda g, be, na: (g, 0)),
    )
    return pl.pallas_call(
        _ffn_body,
        grid_spec=grid_spec,
        out_shape=jax.ShapeDtypeStruct((S, D), jnp.float32),
    )(block_expert, n_active, xg, w1, w3, w2, ws2d)


def _sc_combine(yg, p0, p1, T, D):
    info = plsc.get_sparse_core_info()
    NW = info.num_cores * info.num_subcores
    L = info.num_lanes
    per_w = T // NW
    CH = 8
    n_ch = per_w // CH
    mesh = plsc.VectorSubcoreMesh(core_axis_name="c", subcore_axis_name="s")

    @functools.partial(
        pl.kernel, mesh=mesh,
        out_type=jax.ShapeDtypeStruct((T, D), jnp.float32),
        scratch_types=[
            pltpu.VMEM((per_w,), jnp.int32),
            pltpu.VMEM((per_w,), jnp.int32),
            pltpu.VMEM((CH, D), jnp.float32),
            pltpu.VMEM((CH, D), jnp.float32),
            pltpu.VMEM((CH, D), jnp.float32),
            pltpu.VMEM((CH, D), jnp.float32),
            pltpu.SemaphoreType.DMA,
            pltpu.SemaphoreType.DMA,
            pltpu.SemaphoreType.DMA,
            pltpu.SemaphoreType.DMA,
        ],
    )
    def combine_k(yg_hbm, p0_hbm, p1_hbm, out_hbm, p0_v, p1_v,
                  ba0, bb0, ba1, bb1, sa0, sb0, sa1, sb1):
        nc = info.num_cores
        wid = lax.axis_index("s") * nc + lax.axis_index("c")
        base = wid * per_w
        pltpu.sync_copy(p0_hbm.at[pl.ds(base, per_w)], p0_v)
        pltpu.sync_copy(p1_hbm.at[pl.ds(base, per_w)], p1_v)
        bas, bbs = [ba0, ba1], [bb0, bb1]
        sas, sbs = [sa0, sa1], [sb0, sb1]
        ha, hb = [None, None], [None, None]

        def issue(c):
            j = c % 2
            ha[j] = pltpu.async_copy(
                yg_hbm.at[p0_v.at[pl.ds(c * CH, CH)]], bas[j], sas[j])
            hb[j] = pltpu.async_copy(
                yg_hbm.at[p1_v.at[pl.ds(c * CH, CH)]], bbs[j], sbs[j])

        issue(0)
        for c in range(n_ch):
            j = c % 2
            if c + 1 < n_ch:
                issue(c + 1)
            ha[j].wait()
            hb[j].wait()
            for r in range(CH):
                def add_row(k, _, r=r, j=j):
                    sl = pl.ds(k * L, L)
                    bas[j][r, sl] = bas[j][r, sl] + bbs[j][r, sl]
                    return 0
                lax.fori_loop(0, D // L, add_row, 0)
            pltpu.sync_copy(bas[j], out_hbm.at[pl.ds(base + c * CH, CH)])

    return combine_k(yg, p0, p1)


def kernel(hidden_states, gate_weight, w1, w3, w2):
    T, D = hidden_states.shape
    E = gate_weight.shape[0]
    Gmax = (TOP_K * T) // BT + E
    S = Gmax * BT

    i1, i2, wa, wb, xpack = _router(hidden_states, gate_weight)
    (token_of_slot, weight_of_slot, p0, p1, block_expert,
     n_active) = _metadata(i1[:, 0], i2[:, 0], wa[:, 0], wb[:, 0], E, Gmax, S)
    xg = _sc_gather(xpack, token_of_slot, S)
    yg = _ffn(xg, w1, w3, w2, weight_of_slot, block_expert,
              n_active, Gmax, S)
    return _sc_combine(yg, p0, p1, T, D)
